# baseline (speedup 1.0000x reference)
import functools
import math

import jax
import jax.numpy as jnp
import numpy as np
from jax import lax
from jax.experimental import pallas as pl
from jax.experimental.pallas import tpu as pltpu
from jax.experimental.pallas import tpu_sc as plsc

VOCAB = 1000000
D_MODEL = 64
B = 4096
L = 200
N = B * L

NUM_CORES = 2
NUM_SUBCORES = 16
NUM_WORKERS = NUM_CORES * NUM_SUBCORES
BATCH_BLOCK = B // NUM_WORKERS
LANES = 16
PADDED_D = 128
JGROUPS = D_MODEL // LANES


def _make_pe(max_len, d_model):
    pos = np.arange(max_len, dtype=np.float32)[:, None]
    div = np.exp(
        np.arange(0, d_model, 2, dtype=np.float32) * (-math.log(10000.0) / d_model)
    )
    pe = np.zeros((max_len, d_model), dtype=np.float32)
    pe[:, 0::2] = np.sin(pos * div)
    pe[:, 1::2] = np.cos(pos * div)
    return pe


_PE = _make_pe(L, D_MODEL)


DT = D_MODEL // 8


@functools.partial(
    pl.kernel,
    mesh=plsc.VectorSubcoreMesh(core_axis_name="c", subcore_axis_name="s"),
    compiler_params=pltpu.CompilerParams(
        use_tc_tiling_on_sc=False, needs_layout_passes=False),
    out_type=jax.ShapeDtypeStruct((L, DT, NUM_WORKERS, 8 * BATCH_BLOCK),
                                  jnp.float32),
    scratch_types=[
        pltpu.VMEM((L * D_MODEL,), jnp.float32),
        [pltpu.VMEM((BATCH_BLOCK,), jnp.int32)] * 2,
        [pltpu.VMEM((BATCH_BLOCK, PADDED_D), jnp.float32)] * 2,
        [pltpu.VMEM((D_MODEL * BATCH_BLOCK,), jnp.float32)] * 2,
        [pltpu.SemaphoreType.DMA] * 2,
        [pltpu.SemaphoreType.DMA] * 2,
        [pltpu.SemaphoreType.DMA] * 2,
    ],
)
def _emb_lookup(ids_hbm, table_hbm, pe_hbm, out_hbm,
                pe_v, idx_l, rows_v, slab_v, sem_i, sem_g, sem_o):
    wid = lax.axis_index("s") * NUM_CORES + lax.axis_index("c")
    bbase = wid * BATCH_BLOCK

    pltpu.sync_copy(pe_hbm, pe_v)

    def issue_idx(l, b):
        pltpu.async_copy(ids_hbm.at[pl.ds(l * B + bbase, BATCH_BLOCK)],
                         idx_l[b], sem_i[b])

    def wait_idx(b):
        pltpu.make_async_copy(ids_hbm.at[pl.ds(0, BATCH_BLOCK)],
                              idx_l[b], sem_i[b]).wait()

    def issue_gather(b):
        pltpu.async_copy(table_hbm.at[idx_l[b]], rows_v[b], sem_g[b])

    def wait_gather(b):
        pltpu.make_async_copy(table_hbm.at[pl.ds(0, BATCH_BLOCK)],
                              rows_v[b], sem_g[b]).wait()

    def wait_out(b):
        for dt in range(DT):
            pltpu.make_async_copy(
                slab_v[b].at[pl.ds(dt * 8 * BATCH_BLOCK, 8 * BATCH_BLOCK)],
                out_hbm.at[0, dt, 0], sem_o[b]).wait()

    def process(l, b, next_idx, drain_out):
        wait_gather(b)
        if next_idx is not None:
            issue_idx(next_idx, b)
        if drain_out:
            wait_out(b)

        iota = lax.iota(jnp.int32, LANES)
        carry = tuple(pe_v[pl.ds(l * D_MODEL + j * LANES, LANES)]
                      for j in range(JGROUPS)) + tuple(
                          (iota + j * LANES) * BATCH_BLOCK
                          for j in range(JGROUPS))

        def tok_body(p, c):
            for j in range(JGROUPS):
                vals = rows_v[b][p, pl.ds(j * LANES, LANES)] * 8.0 + c[j]
                plsc.store_scatter(slab_v[b], [c[JGROUPS + j] + p], vals)
            return c

        lax.fori_loop(0, BATCH_BLOCK, tok_body, carry, unroll=False)
        for dt in range(DT):
            pltpu.async_copy(
                slab_v[b].at[pl.ds(dt * 8 * BATCH_BLOCK, 8 * BATCH_BLOCK)],
                out_hbm.at[l, dt, wid], sem_o[b])

    issue_idx(0, 0)
    issue_idx(1, 1)
    wait_idx(0)
    issue_gather(0)
    wait_idx(1)
    issue_gather(1)
    process(0, 0, next_idx=2, drain_out=False)
    wait_idx(0)
    issue_gather(0)
    process(1, 1, next_idx=3, drain_out=False)

    def pair_body(tt, _):
        l0 = tt * 2
        wait_idx(1)
        issue_gather(1)
        process(l0, 0, next_idx=l0 + 2, drain_out=True)
        wait_idx(0)
        issue_gather(0)
        process(l0 + 1, 1, next_idx=l0 + 3, drain_out=True)
        return _

    lax.fori_loop(1, L // 2 - 1, pair_body, None, unroll=False)

    wait_idx(1)
    issue_gather(1)
    process(L - 2, 0, next_idx=None, drain_out=True)
    process(L - 1, 1, next_idx=None, drain_out=True)
    wait_out(0)
    wait_out(1)


FULL_CHUNKS = VOCAB // PADDED_D
REM = VOCAB - FULL_CHUNKS * PADDED_D
CHUNKS_LOW = FULL_CHUNKS // NUM_WORKERS
EXTRA = FULL_CHUNKS - CHUNKS_LOW * NUM_WORKERS


@functools.partial(
    pl.kernel,
    mesh=plsc.VectorSubcoreMesh(core_axis_name="c", subcore_axis_name="s"),
    compiler_params=pltpu.CompilerParams(
        use_tc_tiling_on_sc=True, needs_layout_passes=False),
    out_type=jax.ShapeDtypeStruct((VOCAB, PADDED_D), jnp.float32),
    scratch_types=[
        [pltpu.VMEM((D_MODEL, PADDED_D), jnp.float32)] * 2,
        [pltpu.VMEM((PADDED_D, PADDED_D + 1), jnp.float32)] * 2,
        pltpu.VMEM((D_MODEL, REM), jnp.float32),
        [pltpu.SemaphoreType.DMA] * 2,
        [pltpu.SemaphoreType.DMA] * 2,
    ],
)
def _transpose_table(wt_hbm, wrem_hbm, out_hbm, in_v, tr_v, rem_v,
                     sem_i, sem_o):
    wid = lax.axis_index("s") * NUM_CORES + lax.axis_index("c")

    def issue(c, b):
        off = pl.multiple_of(c * PADDED_D, PADDED_D)
        pltpu.async_copy(wt_hbm.at[:, pl.ds(off, PADDED_D)],
                         in_v[b], sem_i[b])

    def wait_in(b):
        pltpu.make_async_copy(wt_hbm.at[:, pl.ds(0, PADDED_D)],
                              in_v[b], sem_i[b]).wait()

    def wait_out(b):
        pltpu.make_async_copy(tr_v[b].at[:, pl.ds(0, PADDED_D)],
                              out_hbm.at[pl.ds(0, PADDED_D)], sem_o[b]).wait()

    def process(c, b, drain_out):
        wait_in(b)
        if drain_out:
            wait_out(b)
        iota = lax.iota(jnp.int32, LANES)

        def d_body(d, vbases):
            colv = jnp.zeros((LANES,), jnp.int32) + d
            for g in range(PADDED_D // LANES):
                vals = in_v[b][d, pl.ds(g * LANES, LANES)]
                plsc.store_scatter(tr_v[b], [vbases[g], colv], vals)
            return vbases

        lax.fori_loop(0, D_MODEL, d_body,
                      tuple(iota + g * LANES
                            for g in range(PADDED_D // LANES)),
                      unroll=False)
        off = pl.multiple_of(c * PADDED_D, PADDED_D)
        pltpu.async_copy(tr_v[b].at[:, pl.ds(0, PADDED_D)],
                         out_hbm.at[pl.ds(off, PADDED_D)],
                         sem_o[b])

    issue(wid, 0)
    issue(wid + NUM_WORKERS, 1)
    process(wid, 0, drain_out=False)
    issue(wid + 2 * NUM_WORKERS, 0)
    process(wid + NUM_WORKERS, 1, drain_out=False)

    def pair_body(t, _):
        i0 = t * 2

        @pl.when(wid + (i0 + 1) * NUM_WORKERS < FULL_CHUNKS)
        def _issue1():
            issue(wid + (i0 + 1) * NUM_WORKERS, 1)

        @pl.when(wid + i0 * NUM_WORKERS < FULL_CHUNKS)
        def _proc0():
            process(wid + i0 * NUM_WORKERS, 0, drain_out=True)

        @pl.when(wid + (i0 + 2) * NUM_WORKERS < FULL_CHUNKS)
        def _issue2():
            issue(wid + (i0 + 2) * NUM_WORKERS, 0)

        @pl.when(wid + (i0 + 1) * NUM_WORKERS < FULL_CHUNKS)
        def _proc1():
            process(wid + (i0 + 1) * NUM_WORKERS, 1, drain_out=True)
        return _

    lax.fori_loop(1, (CHUNKS_LOW + 2) // 2 + 1, pair_body, None,
                  unroll=False)
    wait_out(0)
    wait_out(1)

    @pl.when(wid == NUM_WORKERS - 1)
    def _rem():
        pltpu.sync_copy(wrem_hbm, rem_v)
        iota = lax.iota(jnp.int32, LANES)

        def d_body(d, vbases):
            colv = jnp.zeros((LANES,), jnp.int32) + d
            for g in range(REM // LANES):
                vals = rem_v[d, pl.ds(g * LANES, LANES)]
                plsc.store_scatter(tr_v[0], [vbases[g], colv], vals)
            return vbases

        lax.fori_loop(0, D_MODEL, d_body,
                      tuple(iota + g * LANES for g in range(REM // LANES)),
                      unroll=False)
        pltpu.sync_copy(tr_v[0].at[pl.ds(0, REM), pl.ds(0, PADDED_D)],
                        out_hbm.at[pl.ds(FULL_CHUNKS * PADDED_D, REM)])


def kernel(input_ids, W):
    ids_lmajor = input_ids.T.reshape(-1).astype(jnp.int32)
    table = _transpose_table(W.T, W.T[:, FULL_CHUNKS * PADDED_D:])
    out = _emb_lookup(ids_lmajor, table, jnp.asarray(_PE).reshape(-1))
    out5 = out.reshape(L, DT, NUM_WORKERS, 8, BATCH_BLOCK)
    return out5.transpose(2, 4, 0, 1, 3).reshape(B, L, D_MODEL)


# speedup vs baseline: 1.0082x; 1.0082x over previous
import functools
import math

import jax
import jax.numpy as jnp
import numpy as np
from jax import lax
from jax.experimental import pallas as pl
from jax.experimental.pallas import tpu as pltpu
from jax.experimental.pallas import tpu_sc as plsc

VOCAB = 1000000
D_MODEL = 64
B = 4096
L = 200
N = B * L

NUM_CORES = 2
NUM_SUBCORES = 16
NUM_WORKERS = NUM_CORES * NUM_SUBCORES
BATCH_BLOCK = B // NUM_WORKERS
LANES = 16
PADDED_D = 128
JGROUPS = D_MODEL // LANES


def _make_pe(max_len, d_model):
    pos = np.arange(max_len, dtype=np.float32)[:, None]
    div = np.exp(
        np.arange(0, d_model, 2, dtype=np.float32) * (-math.log(10000.0) / d_model)
    )
    pe = np.zeros((max_len, d_model), dtype=np.float32)
    pe[:, 0::2] = np.sin(pos * div)
    pe[:, 1::2] = np.cos(pos * div)
    return pe


_PE = _make_pe(L, D_MODEL)


DT = D_MODEL // 8


@functools.partial(
    pl.kernel,
    mesh=plsc.VectorSubcoreMesh(core_axis_name="c", subcore_axis_name="s"),
    compiler_params=pltpu.CompilerParams(
        use_tc_tiling_on_sc=False, needs_layout_passes=False),
    out_type=jax.ShapeDtypeStruct((L, DT, NUM_WORKERS, 8 * BATCH_BLOCK),
                                  jnp.float32),
    scratch_types=[
        pltpu.VMEM((L * D_MODEL,), jnp.float32),
        [pltpu.VMEM((BATCH_BLOCK,), jnp.int32)] * 2,
        [pltpu.VMEM((BATCH_BLOCK, PADDED_D), jnp.float32)] * 2,
        [pltpu.VMEM((DT, 8 * BATCH_BLOCK), jnp.float32)] * 2,
        [pltpu.SemaphoreType.DMA] * 2,
        [pltpu.SemaphoreType.DMA] * 2,
        [pltpu.SemaphoreType.DMA] * 2,
    ],
)
def _emb_lookup(ids_hbm, table_hbm, pe_hbm, out_hbm,
                pe_v, idx_l, rows_v, slab_v, sem_i, sem_g, sem_o):
    wid = lax.axis_index("s") * NUM_CORES + lax.axis_index("c")
    bbase = wid * BATCH_BLOCK

    pltpu.sync_copy(pe_hbm, pe_v)

    def issue_idx(l, b):
        pltpu.async_copy(ids_hbm.at[pl.ds(l * B + bbase, BATCH_BLOCK)],
                         idx_l[b], sem_i[b])

    def wait_idx(b):
        pltpu.make_async_copy(ids_hbm.at[pl.ds(0, BATCH_BLOCK)],
                              idx_l[b], sem_i[b]).wait()

    def issue_gather(b):
        pltpu.async_copy(table_hbm.at[idx_l[b]], rows_v[b], sem_g[b])

    def wait_gather(b):
        pltpu.make_async_copy(table_hbm.at[pl.ds(0, BATCH_BLOCK)],
                              rows_v[b], sem_g[b]).wait()

    def wait_out(b):
        pltpu.make_async_copy(
            slab_v[b], out_hbm.at[0, :, 0], sem_o[b]).wait()

    def process(l, b, next_idx, drain_out):
        wait_gather(b)
        if next_idx is not None:
            issue_idx(next_idx, b)
        if drain_out:
            wait_out(b)

        iota = lax.iota(jnp.int32, LANES)
        dvs = tuple(iota + j * LANES for j in range(JGROUPS))
        carry = tuple(pe_v[pl.ds(l * D_MODEL + j * LANES, LANES)]
                      for j in range(JGROUPS)) + tuple(
                          dv >> 3 for dv in dvs) + tuple(
                          (dv & 7) << 7 for dv in dvs)

        def tok_body(p, c):
            for j in range(JGROUPS):
                vals = rows_v[b][p, pl.ds(j * LANES, LANES)] * 8.0 + c[j]
                plsc.store_scatter(
                    slab_v[b],
                    [c[JGROUPS + j], c[2 * JGROUPS + j] + p], vals)
            return c

        lax.fori_loop(0, BATCH_BLOCK, tok_body, carry, unroll=8)
        pltpu.async_copy(slab_v[b], out_hbm.at[l, :, wid], sem_o[b])

    issue_idx(0, 0)
    issue_idx(1, 1)
    wait_idx(0)
    issue_gather(0)
    wait_idx(1)
    issue_gather(1)
    process(0, 0, next_idx=2, drain_out=False)
    wait_idx(0)
    issue_gather(0)
    process(1, 1, next_idx=3, drain_out=False)

    def pair_body(tt, _):
        l0 = tt * 2
        wait_idx(1)
        issue_gather(1)
        process(l0, 0, next_idx=l0 + 2, drain_out=True)
        wait_idx(0)
        issue_gather(0)
        process(l0 + 1, 1, next_idx=l0 + 3, drain_out=True)
        return _

    lax.fori_loop(1, L // 2 - 1, pair_body, None, unroll=False)

    wait_idx(1)
    issue_gather(1)
    process(L - 2, 0, next_idx=None, drain_out=True)
    process(L - 1, 1, next_idx=None, drain_out=True)
    wait_out(0)
    wait_out(1)


FULL_CHUNKS = VOCAB // PADDED_D
REM = VOCAB - FULL_CHUNKS * PADDED_D
CHUNKS_LOW = FULL_CHUNKS // NUM_WORKERS
EXTRA = FULL_CHUNKS - CHUNKS_LOW * NUM_WORKERS


@functools.partial(
    pl.kernel,
    mesh=plsc.VectorSubcoreMesh(core_axis_name="c", subcore_axis_name="s"),
    compiler_params=pltpu.CompilerParams(
        use_tc_tiling_on_sc=True, needs_layout_passes=False),
    out_type=jax.ShapeDtypeStruct((VOCAB, PADDED_D), jnp.float32),
    scratch_types=[
        [pltpu.VMEM((D_MODEL, PADDED_D), jnp.float32)] * 2,
        [pltpu.VMEM((PADDED_D, PADDED_D), jnp.float32)] * 2,
        pltpu.VMEM((D_MODEL, REM), jnp.float32),
        [pltpu.SemaphoreType.DMA] * 2,
        [pltpu.SemaphoreType.DMA] * 2,
    ],
)
def _transpose_table(wt_hbm, wrem_hbm, out_hbm, in_v, tr_v, rem_v,
                     sem_i, sem_o):
    wid = lax.axis_index("s") * NUM_CORES + lax.axis_index("c")

    def issue(c, b):
        off = pl.multiple_of(c * PADDED_D, PADDED_D)
        pltpu.async_copy(wt_hbm.at[:, pl.ds(off, PADDED_D)],
                         in_v[b], sem_i[b])

    def wait_in(b):
        pltpu.make_async_copy(wt_hbm.at[:, pl.ds(0, PADDED_D)],
                              in_v[b], sem_i[b]).wait()

    def wait_out(b):
        pltpu.make_async_copy(tr_v[b],
                              out_hbm.at[pl.ds(0, PADDED_D)], sem_o[b]).wait()

    def process(c, b, drain_out):
        wait_in(b)
        if drain_out:
            wait_out(b)
        iota = lax.iota(jnp.int32, LANES)

        def d_body(d, vbases):
            colv = jnp.zeros((LANES,), jnp.int32) + d
            for g in range(PADDED_D // LANES):
                vals = in_v[b][d, pl.ds(g * LANES, LANES)]
                plsc.store_scatter(tr_v[b], [vbases[g], colv], vals)
            return vbases

        lax.fori_loop(0, D_MODEL, d_body,
                      tuple(iota + g * LANES
                            for g in range(PADDED_D // LANES)),
                      unroll=4)
        off = pl.multiple_of(c * PADDED_D, PADDED_D)
        pltpu.async_copy(tr_v[b], out_hbm.at[pl.ds(off, PADDED_D)],
                         sem_o[b])

    issue(wid, 0)
    issue(wid + NUM_WORKERS, 1)
    process(wid, 0, drain_out=False)
    issue(wid + 2 * NUM_WORKERS, 0)
    process(wid + NUM_WORKERS, 1, drain_out=False)

    def pair_body(t, _):
        i0 = t * 2

        @pl.when(wid + (i0 + 1) * NUM_WORKERS < FULL_CHUNKS)
        def _issue1():
            issue(wid + (i0 + 1) * NUM_WORKERS, 1)

        @pl.when(wid + i0 * NUM_WORKERS < FULL_CHUNKS)
        def _proc0():
            process(wid + i0 * NUM_WORKERS, 0, drain_out=True)

        @pl.when(wid + (i0 + 2) * NUM_WORKERS < FULL_CHUNKS)
        def _issue2():
            issue(wid + (i0 + 2) * NUM_WORKERS, 0)

        @pl.when(wid + (i0 + 1) * NUM_WORKERS < FULL_CHUNKS)
        def _proc1():
            process(wid + (i0 + 1) * NUM_WORKERS, 1, drain_out=True)
        return _

    lax.fori_loop(1, (CHUNKS_LOW + 2) // 2 + 1, pair_body, None,
                  unroll=False)
    wait_out(0)
    wait_out(1)

    @pl.when(wid == NUM_WORKERS - 1)
    def _rem():
        pltpu.sync_copy(wrem_hbm, rem_v)
        iota = lax.iota(jnp.int32, LANES)

        def d_body(d, vbases):
            colv = jnp.zeros((LANES,), jnp.int32) + d
            for g in range(REM // LANES):
                vals = rem_v[d, pl.ds(g * LANES, LANES)]
                plsc.store_scatter(tr_v[0], [vbases[g], colv], vals)
            return vbases

        lax.fori_loop(0, D_MODEL, d_body,
                      tuple(iota + g * LANES for g in range(REM // LANES)),
                      unroll=False)
        pltpu.sync_copy(tr_v[0].at[pl.ds(0, REM)],
                        out_hbm.at[pl.ds(FULL_CHUNKS * PADDED_D, REM)])


def kernel(input_ids, W):
    ids_lmajor = input_ids.T.reshape(-1).astype(jnp.int32)
    table = _transpose_table(W.T, W.T[:, FULL_CHUNKS * PADDED_D:])
    out = _emb_lookup(ids_lmajor, table, jnp.asarray(_PE).reshape(-1))
    out5 = out.reshape(L, DT, NUM_WORKERS, 8, BATCH_BLOCK)
    return out5.transpose(2, 4, 0, 1, 3).reshape(B, L, D_MODEL)


# speedup vs baseline: 2.0375x; 2.0210x over previous
import functools
import math

import jax
import jax.numpy as jnp
import numpy as np
from jax import lax
from jax.experimental import pallas as pl
from jax.experimental.pallas import tpu as pltpu
from jax.experimental.pallas import tpu_sc as plsc

VOCAB = 1000000
D_MODEL = 64
B = 4096
L = 200
N = B * L

NUM_CORES = 2
NUM_SUBCORES = 16
NUM_WORKERS = NUM_CORES * NUM_SUBCORES
ROWS_PER_WORKER = N // NUM_WORKERS
NCHUNK = ROWS_PER_WORKER // L
LANES = 16
PADDED_D = 128
SPLIT_A = 104
SPLIT_B = L - SPLIT_A


def _make_pe(max_len, d_model):
    pos = np.arange(max_len, dtype=np.float32)[:, None]
    div = np.exp(
        np.arange(0, d_model, 2, dtype=np.float32) * (-math.log(10000.0) / d_model)
    )
    pe = np.zeros((max_len, d_model), dtype=np.float32)
    pe[:, 0::2] = np.sin(pos * div)
    pe[:, 1::2] = np.cos(pos * div)
    return pe


_PE = _make_pe(L, D_MODEL)


@functools.partial(
    pl.kernel,
    mesh=plsc.VectorSubcoreMesh(core_axis_name="c", subcore_axis_name="s"),
    compiler_params=pltpu.CompilerParams(use_tc_tiling_on_sc=True),
    out_type=jax.ShapeDtypeStruct((N, D_MODEL), jnp.float32),
    scratch_types=[
        pltpu.VMEM((ROWS_PER_WORKER // 2,), jnp.int32),
        [pltpu.VMEM((L, PADDED_D), jnp.float32)] * 2,
        [pltpu.VMEM((L, D_MODEL), jnp.float32)] * 2,
        pltpu.VMEM((L * D_MODEL,), jnp.float32),
        [pltpu.SemaphoreType.DMA] * 2,
        [pltpu.SemaphoreType.DMA] * 2,
        [pltpu.SemaphoreType.DMA] * 2,
    ],
)
def _emb_lookup(ids_hbm, table_hbm, pe_hbm, out_hbm,
                idx_v, rows_v, stage_v, pe_v, sem_ga, sem_gb, sem_out):
    wid = lax.axis_index("s") * NUM_CORES + lax.axis_index("c")
    base = wid * ROWS_PER_WORKER

    pltpu.sync_copy(pe_hbm, pe_v)

    def issue_gather(g, b):
        loc = g * L
        pltpu.async_copy(
            table_hbm.at[idx_v.at[pl.ds(loc, SPLIT_A)]],
            rows_v[b].at[pl.ds(0, SPLIT_A)], sem_ga[b])
        pltpu.async_copy(
            table_hbm.at[idx_v.at[pl.ds(loc + SPLIT_A, SPLIT_B)]],
            rows_v[b].at[pl.ds(SPLIT_A, SPLIT_B)], sem_gb[b])

    def wait_gather(b):
        pltpu.make_async_copy(
            table_hbm.at[pl.ds(0, SPLIT_A)],
            rows_v[b].at[pl.ds(0, SPLIT_A)], sem_ga[b]).wait()
        pltpu.make_async_copy(
            table_hbm.at[pl.ds(0, SPLIT_B)],
            rows_v[b].at[pl.ds(SPLIT_A, SPLIT_B)], sem_gb[b]).wait()

    def wait_out(b):
        pltpu.make_async_copy(
            stage_v[b], out_hbm.at[pl.ds(0, L)], sem_out[b]).wait()

    HCHUNK = NCHUNK // 2

    def make_process(half_base):
        def process(g, b, drain_out):
            wait_gather(b)
            if drain_out:
                wait_out(b)

            def row_body(p, _):
                o = p * D_MODEL
                for j in range(D_MODEL // LANES):
                    c = j * LANES
                    stage_v[b][p, pl.ds(c, LANES)] = (
                        rows_v[b][p, pl.ds(c, LANES)] * 8.0
                        + pe_v[pl.ds(o + c, LANES)])
                return _

            lax.fori_loop(0, L, row_body, None, unroll=False)
            off = pl.multiple_of(half_base + g * L, L)
            pltpu.async_copy(stage_v[b], out_hbm.at[pl.ds(off, L)],
                             sem_out[b])
        return process

    for h in range(2):
        half_base = base + h * HCHUNK * L
        pltpu.sync_copy(
            ids_hbm.at[pl.ds(half_base, HCHUNK * L)], idx_v)
        process = make_process(half_base)

        issue_gather(0, 0)
        issue_gather(1, 1)
        process(0, 0, drain_out=False)
        issue_gather(2, 0)
        process(1, 1, drain_out=False)
        issue_gather(3, 1)

        def pair_body(tt, _):
            g0 = tt * 2
            process(g0, 0, drain_out=True)
            issue_gather(g0 + 2, 0)
            process(g0 + 1, 1, drain_out=True)
            issue_gather(g0 + 3, 1)
            return _

        lax.fori_loop(1, HCHUNK // 2 - 1, pair_body, None, unroll=False)

        process(HCHUNK - 2, 0, drain_out=True)
        process(HCHUNK - 1, 1, drain_out=True)
        wait_out(0)
        wait_out(1)


def kernel(input_ids, W):
    ids_flat = input_ids.reshape(-1).astype(jnp.int32)
    table = jnp.pad(W, ((0, 0), (0, PADDED_D - D_MODEL)))
    out = _emb_lookup(ids_flat, table, jnp.asarray(_PE).reshape(-1))
    return out.reshape(B, L, D_MODEL)
